# baseline (speedup 1.0000x reference)
import functools

import jax
import jax.numpy as jnp
from jax import lax
from jax.experimental import pallas as pl
from jax.experimental.pallas import tpu as pltpu
from jax.experimental.pallas import tpu_sc as plsc

N = 10000
D = 128
H = 128
E = 320000
OUT = 40

NC = 2
NS = 16
NW = NC * NS

CK = 128
NBUF = 2
CH_BIG = 80
CH_SMALL = 80
BIG_ON_C0 = True
SEG = 80
TOT_CH = NS * (CH_BIG + CH_SMALL)
E_PAD = TOT_CH * CK
ACC_ROWS = 10240
ZROWS_W = ACC_ROWS // NS

BROWS = 2000


def _make_segsum(want_deg: bool):
    mesh = plsc.VectorSubcoreMesh(core_axis_name="c", subcore_axis_name="s")
    out_type = [jax.ShapeDtypeStruct((NC, ACC_ROWS, H), jnp.float32)]
    scratch = [
        pltpu.VMEM((SEG * CK,), jnp.int32),
        [pltpu.VMEM((1, CK), jnp.int32) for _ in range(NBUF)],
        [pltpu.VMEM((CK, H), jnp.float32) for _ in range(NBUF)],
        pltpu.VMEM_SHARED((ACC_ROWS, H), jnp.float32),
        [pltpu.SemaphoreType.DMA for _ in range(NBUF)],
        [pltpu.SemaphoreType.DMA for _ in range(NBUF)],
    ]
    if want_deg:
        out_type.append(jax.ShapeDtypeStruct((ACC_ROWS,), jnp.float32))
        out_type.append(jax.ShapeDtypeStruct((ACC_ROWS,), jnp.float32))
        scratch += [
            pltpu.VMEM((CK,), jnp.float32),
            pltpu.VMEM_SHARED((ACC_ROWS,), jnp.float32),
        ]

    def body(feat, srcm, dstm, zrows, zdeg, *rest):
        if want_deg:
            (agg_out, deg0_out, deg1_out, src_v, dsts, rows, acc_sh, sems,
             dsems, ones_v, deg_sh) = rest
        else:
            agg_out, src_v, dsts, rows, acc_sh, sems, dsems = rest
        c = lax.axis_index("c")
        s = lax.axis_index("s")
        z0 = s * ZROWS_W
        big = (c == 0) if BIG_ON_C0 else (c == 1)
        ch = jnp.where(big, CH_BIG, CH_SMALL)
        slab = jnp.where(big, s * CH_BIG, NS * CH_BIG + s * CH_SMALL)
        pltpu.sync_copy(zrows.at[pl.ds(z0, ZROWS_W)], acc_sh.at[pl.ds(z0, ZROWS_W)])
        if want_deg:
            pltpu.sync_copy(zdeg.at[pl.ds(z0, ZROWS_W)], deg_sh.at[pl.ds(z0, ZROWS_W)])
            for j in range(CK // 16):
                ones_v[pl.ds(j * 16, 16)] = jnp.ones((16,), jnp.float32)
        plsc.subcore_barrier()

        bufs = tuple(zip(rows, sems, dsts, dsems))

        def _consume(buf, dbuf):
            pltpu.sync_copy(buf, acc_sh.at[dbuf.at[0]], add=True)
            if want_deg:
                pltpu.sync_copy(ones_v, deg_sh.at[dbuf.at[0]], add=True)

        def run_segment(first, nch):
            pltpu.sync_copy(
                srcm.at[pl.ds(pl.multiple_of(first * CK, 128), SEG * CK)],
                src_v)

            def _start(j, buf, sem, dbuf, dsem):
                pltpu.async_copy(dstm.at[first + j], dbuf, dsem)
                pltpu.async_copy(feat.at[src_v.at[pl.ds(j * CK, CK)]], buf, sem)

            def _drain(buf, sem, dbuf, dsem):
                pltpu.make_async_copy(dstm.at[0], dbuf, dsem).wait()
                pltpu.make_async_copy(
                    feat.at[src_v.at[pl.ds(0, CK)]], buf, sem).wait()

            for b, (buf, sem, dbuf, dsem) in enumerate(bufs):
                _start(b, buf, sem, dbuf, dsem)

            def step(g, carry):
                for b, (buf, sem, dbuf, dsem) in enumerate(bufs):
                    j = g * NBUF + b
                    _drain(buf, sem, dbuf, dsem)
                    _consume(buf, dbuf)
                    _start(j + NBUF, buf, sem, dbuf, dsem)
                return carry

            lax.fori_loop(0, nch // NBUF - 1, step, 0)
            for b, (buf, sem, dbuf, dsem) in enumerate(bufs):
                _drain(buf, sem, dbuf, dsem)
                _consume(buf, dbuf)

        run_segment(slab, jnp.minimum(ch, SEG))

        @pl.when(ch > SEG)
        def _():
            run_segment(slab + SEG, ch - SEG)

        plsc.subcore_barrier()
        pltpu.sync_copy(acc_sh.at[pl.ds(z0, ZROWS_W)], agg_out.at[c, pl.ds(z0, ZROWS_W)])
        if want_deg:
            @pl.when(c == 0)
            def _():
                pltpu.sync_copy(deg_sh.at[pl.ds(z0, ZROWS_W)],
                                deg0_out.at[pl.ds(z0, ZROWS_W)])

            @pl.when(c == 1)
            def _():
                pltpu.sync_copy(deg_sh.at[pl.ds(z0, ZROWS_W)],
                                deg1_out.at[pl.ds(z0, ZROWS_W)])

    return pl.kernel(body, out_type=tuple(out_type), mesh=mesh,
                     scratch_types=tuple(scratch))


_segsum_deg = _make_segsum(True)
_segsum = _make_segsum(False)


def _tc_layer_body(h, a0, a1, d0, d1, Wl, Wr, bl, g, beta, o_ref):
    deg = jnp.maximum(d0[...] + d1[...], 1.0)
    mean = (a0[0] + a1[0]) / deg
    o = jnp.dot(mean, Wl[...], preferred_element_type=jnp.float32)
    o = o + jnp.dot(h[...], Wr[...], preferred_element_type=jnp.float32)
    o = o + bl[...]
    o = jnp.maximum(o, 0.0)
    mu = jnp.mean(o, axis=1, keepdims=True)
    var = jnp.mean((o - mu) ** 2, axis=1, keepdims=True)
    o_ref[...] = (o - mu) * lax.rsqrt(var + 1e-5) * g[...] + beta[...]


def _tc_layer(h, agg, d0, d1, Wl, Wr, bl, g, beta):
    grid = (N // BROWS,)
    bs_rows = pl.BlockSpec((BROWS, H), lambda i: (i, 0))
    bs_a0 = pl.BlockSpec((1, BROWS, H), lambda i: (0, i, 0))
    bs_a1 = pl.BlockSpec((1, BROWS, H), lambda i: (1, i, 0))
    bs_col = pl.BlockSpec((BROWS, 1), lambda i: (i, 0))
    bs_w = pl.BlockSpec((H, H), lambda i: (0, 0))
    bs_v = pl.BlockSpec((1, H), lambda i: (0, 0))
    return pl.pallas_call(
        _tc_layer_body,
        grid=grid,
        in_specs=[bs_rows, bs_a0, bs_a1, bs_col, bs_col, bs_w, bs_w,
                  bs_v, bs_v, bs_v],
        out_specs=bs_rows,
        out_shape=jax.ShapeDtypeStruct((N, H), jnp.float32),
    )(h, agg, agg, d0, d1, Wl, Wr, bl, g, beta)


def _tc_final_body(h, a0, a1, d0, d1, Wl, Wr, bl, W1, b1, W2, b2,
                   emb_ref, out_ref):
    deg = jnp.maximum(d0[...] + d1[...], 1.0)
    mean = (a0[0] + a1[0]) / deg
    o = jnp.dot(mean, Wl[...], preferred_element_type=jnp.float32)
    o = o + jnp.dot(h[...], Wr[...], preferred_element_type=jnp.float32)
    o = o + bl[...]
    o = jnp.maximum(o, 0.0)
    emb_ref[...] = o
    t = jnp.maximum(jnp.dot(o, W1[...], preferred_element_type=jnp.float32)
                    + b1[...], 0.0)
    logits = jnp.dot(t, W2[...], preferred_element_type=jnp.float32) + b2[...]
    m = jnp.max(logits, axis=1, keepdims=True)
    ex = jnp.exp(logits - m)
    lse = jnp.log(jnp.sum(ex, axis=1, keepdims=True)) + m
    out_ref[...] = logits - lse


def _tc_final(h, agg, d0, d1, Wl, Wr, bl, W1p, b1p, W2p, b2):
    P1 = W1p.shape[1]
    grid = (N // BROWS,)
    bs_rows = pl.BlockSpec((BROWS, H), lambda i: (i, 0))
    bs_a0 = pl.BlockSpec((1, BROWS, H), lambda i: (0, i, 0))
    bs_a1 = pl.BlockSpec((1, BROWS, H), lambda i: (1, i, 0))
    bs_col = pl.BlockSpec((BROWS, 1), lambda i: (i, 0))
    bs_w = pl.BlockSpec((H, H), lambda i: (0, 0))
    bs_v = pl.BlockSpec((1, H), lambda i: (0, 0))
    return pl.pallas_call(
        _tc_final_body,
        grid=grid,
        in_specs=[bs_rows, bs_a0, bs_a1, bs_col, bs_col, bs_w, bs_w, bs_v,
                  pl.BlockSpec((H, P1), lambda i: (0, 0)),
                  pl.BlockSpec((1, P1), lambda i: (0, 0)),
                  pl.BlockSpec((P1, OUT), lambda i: (0, 0)),
                  pl.BlockSpec((1, OUT), lambda i: (0, 0))],
        out_specs=[bs_rows, pl.BlockSpec((BROWS, OUT), lambda i: (i, 0))],
        out_shape=[jax.ShapeDtypeStruct((N, H), jnp.float32),
                   jax.ShapeDtypeStruct((N, OUT), jnp.float32)],
    )(h, agg, agg, d0, d1, Wl, Wr, bl, W1p, b1p, W2p, b2)


def kernel(x, edge_index, batch, Wl0, bl0, Wr0, Wl1, bl1, Wr1, Wl2, bl2, Wr2,
           g0, beta0, g1, beta1, W1, b1, W2, b2):
    src = edge_index[0]
    dst = edge_index[1]
    npad = E_PAD - E
    srcm = jnp.concatenate(
        [src, jnp.zeros((npad + SEG * CK,), jnp.int32)])
    pad_dst = N + (jnp.arange(npad, dtype=jnp.int32) % (ACC_ROWS - N))
    dstm = jnp.concatenate([dst, pad_dst]).reshape(TOT_CH, 1, CK)
    zrows = jnp.zeros((ACC_ROWS, H), jnp.float32)
    zdeg = jnp.zeros((ACC_ROWS,), jnp.float32)

    bl0_ = bl0.reshape(1, H)
    bl1_ = bl1.reshape(1, H)
    bl2_ = bl2.reshape(1, H)
    g0_ = g0.reshape(1, H)
    beta0_ = beta0.reshape(1, H)
    g1_ = g1.reshape(1, H)
    beta1_ = beta1.reshape(1, H)
    P1 = 64
    W1p = jnp.pad(W1, ((0, 0), (0, P1 - W1.shape[1])))
    b1p = jnp.pad(b1, (0, P1 - b1.shape[0])).reshape(1, P1)
    W2p = jnp.pad(W2, ((0, P1 - W2.shape[0]), (0, 0)))
    b2_ = b2.reshape(1, OUT)

    agg, deg0, deg1 = _segsum_deg(x, srcm, dstm, zrows, zdeg)
    d0 = deg0.reshape(ACC_ROWS, 1)
    d1 = deg1.reshape(ACC_ROWS, 1)
    h = _tc_layer(x, agg, d0, d1, Wl0, Wr0, bl0_, g0_, beta0_)
    (agg,) = _segsum(h, srcm, dstm, zrows, zdeg)
    h = _tc_layer(h, agg, d0, d1, Wl1, Wr1, bl1_, g1_, beta1_)
    (agg,) = _segsum(h, srcm, dstm, zrows, zdeg)
    emb, logp = _tc_final(h, agg, d0, d1, Wl2, Wr2, bl2_,
                          W1p, b1p, W2p, b2_)
    return (emb, logp)


# speedup vs baseline: 3.6547x; 3.6547x over previous
import functools

import jax
import jax.numpy as jnp
from jax import lax
from jax.experimental import pallas as pl
from jax.experimental.pallas import tpu as pltpu
from jax.experimental.pallas import tpu_sc as plsc

N = 10000
D = 128
H = 128
E = 320000
OUT = 40

NC = 2
NS = 16
NW = NC * NS

CK = 128
NBUF = 2
CH_BIG = 80
CH_SMALL = 80
BIG_ON_C0 = True
SEG = 80
TOT_CH = NS * (CH_BIG + CH_SMALL)
E_PAD = TOT_CH * CK
ACC_ROWS = 10240
ZROWS_W = ACC_ROWS // NS

BROWS = 2000


def _make_segsum(want_deg: bool):
    mesh = plsc.VectorSubcoreMesh(core_axis_name="c", subcore_axis_name="s")
    out_type = [jax.ShapeDtypeStruct((NC, ACC_ROWS, H), jnp.float32)]
    scratch = [
        pltpu.VMEM((SEG * CK,), jnp.int32),
        [pltpu.VMEM((1, CK), jnp.int32) for _ in range(NBUF)],
        [pltpu.VMEM((CK, H), jnp.float32) for _ in range(NBUF)],
        pltpu.VMEM_SHARED((ACC_ROWS, H), jnp.float32),
        [pltpu.SemaphoreType.DMA for _ in range(NBUF)],
        [pltpu.SemaphoreType.DMA for _ in range(NBUF)],
    ]
    if want_deg:
        out_type.append(jax.ShapeDtypeStruct((ACC_ROWS,), jnp.float32))
        out_type.append(jax.ShapeDtypeStruct((ACC_ROWS,), jnp.float32))
        scratch += [
            pltpu.VMEM((CK,), jnp.float32),
            pltpu.VMEM_SHARED((ACC_ROWS,), jnp.float32),
        ]

    def body(feat, srcm, dstm, zrows, zdeg, *rest):
        if want_deg:
            (agg_out, deg0_out, deg1_out, src_v, dsts, rows, acc_sh, sems,
             dsems, ones_v, deg_sh) = rest
        else:
            agg_out, src_v, dsts, rows, acc_sh, sems, dsems = rest
        c = lax.axis_index("c")
        s = lax.axis_index("s")
        z0 = s * ZROWS_W
        big = (c == 0) if BIG_ON_C0 else (c == 1)
        ch = jnp.where(big, CH_BIG, CH_SMALL)
        slab = jnp.where(big, s * CH_BIG, NS * CH_BIG + s * CH_SMALL)
        pltpu.sync_copy(zrows.at[pl.ds(z0, ZROWS_W)], acc_sh.at[pl.ds(z0, ZROWS_W)])
        if want_deg:
            pltpu.sync_copy(zdeg.at[pl.ds(z0, ZROWS_W)], deg_sh.at[pl.ds(z0, ZROWS_W)])
            for j in range(CK // 16):
                ones_v[pl.ds(j * 16, 16)] = jnp.ones((16,), jnp.float32)
        plsc.subcore_barrier()

        bufs = tuple(zip(rows, sems, dsts, dsems))

        def _consume(buf, dbuf):
            pltpu.sync_copy(buf, acc_sh.at[dbuf.at[0]], add=True)
            if want_deg:
                pltpu.sync_copy(ones_v, deg_sh.at[dbuf.at[0]], add=True)

        def run_segment(first, nch):
            pltpu.sync_copy(
                srcm.at[pl.ds(pl.multiple_of(first * CK, 128), SEG * CK)],
                src_v)

            def _start(j, buf, sem, dbuf, dsem):
                pltpu.async_copy(dstm.at[first + j], dbuf, dsem)
                pltpu.async_copy(feat.at[src_v.at[pl.ds(j * CK, CK)]], buf, sem)

            def _drain(buf, sem, dbuf, dsem):
                pltpu.make_async_copy(dstm.at[0], dbuf, dsem).wait()
                pltpu.make_async_copy(
                    feat.at[src_v.at[pl.ds(0, CK)]], buf, sem).wait()

            for b, (buf, sem, dbuf, dsem) in enumerate(bufs):
                _start(b, buf, sem, dbuf, dsem)

            def step(g, carry):
                for b, (buf, sem, dbuf, dsem) in enumerate(bufs):
                    j = g * NBUF + b
                    _drain(buf, sem, dbuf, dsem)
                    _consume(buf, dbuf)
                    _start(j + NBUF, buf, sem, dbuf, dsem)
                return carry

            lax.fori_loop(0, nch // NBUF - 1, step, 0)
            for b, (buf, sem, dbuf, dsem) in enumerate(bufs):
                _drain(buf, sem, dbuf, dsem)
                _consume(buf, dbuf)

        run_segment(slab, jnp.minimum(ch, SEG))

        @pl.when(ch > SEG)
        def _():
            run_segment(slab + SEG, ch - SEG)

        plsc.subcore_barrier()
        pltpu.sync_copy(acc_sh.at[pl.ds(z0, ZROWS_W)], agg_out.at[c, pl.ds(z0, ZROWS_W)])
        if want_deg:
            @pl.when(c == 0)
            def _():
                pltpu.sync_copy(deg_sh.at[pl.ds(z0, ZROWS_W)],
                                deg0_out.at[pl.ds(z0, ZROWS_W)])

            @pl.when(c == 1)
            def _():
                pltpu.sync_copy(deg_sh.at[pl.ds(z0, ZROWS_W)],
                                deg1_out.at[pl.ds(z0, ZROWS_W)])

    return pl.kernel(body, out_type=tuple(out_type), mesh=mesh,
                     scratch_types=tuple(scratch))


_segsum_deg = _make_segsum(True)
_segsum = _make_segsum(False)


def _tc_layer_body(h, a0, a1, d0, d1, Wl, Wr, bl, g, beta, o_ref):
    deg = jnp.maximum(d0[...] + d1[...], 1.0)
    mean = (a0[0] + a1[0]) / deg
    o = jnp.dot(mean, Wl[...], preferred_element_type=jnp.float32)
    o = o + jnp.dot(h[...], Wr[...], preferred_element_type=jnp.float32)
    o = o + bl[...]
    o = jnp.maximum(o, 0.0)
    mu = jnp.mean(o, axis=1, keepdims=True)
    var = jnp.mean((o - mu) ** 2, axis=1, keepdims=True)
    o_ref[...] = (o - mu) * lax.rsqrt(var + 1e-5) * g[...] + beta[...]


def _tc_layer(h, agg, d0, d1, Wl, Wr, bl, g, beta):
    grid = (N // BROWS,)
    bs_rows = pl.BlockSpec((BROWS, H), lambda i: (i, 0))
    bs_a0 = pl.BlockSpec((1, BROWS, H), lambda i: (0, i, 0))
    bs_a1 = pl.BlockSpec((1, BROWS, H), lambda i: (1, i, 0))
    bs_col = pl.BlockSpec((BROWS, 1), lambda i: (i, 0))
    bs_w = pl.BlockSpec((H, H), lambda i: (0, 0))
    bs_v = pl.BlockSpec((1, H), lambda i: (0, 0))
    return pl.pallas_call(
        _tc_layer_body,
        grid=grid,
        in_specs=[bs_rows, bs_a0, bs_a1, bs_col, bs_col, bs_w, bs_w,
                  bs_v, bs_v, bs_v],
        out_specs=bs_rows,
        out_shape=jax.ShapeDtypeStruct((N, H), jnp.float32),
    )(h, agg, agg, d0, d1, Wl, Wr, bl, g, beta)


def _tc_final_body(h, a0, a1, d0, d1, Wl, Wr, bl, W1, b1, W2, b2,
                   emb_ref, out_ref):
    deg = jnp.maximum(d0[...] + d1[...], 1.0)
    mean = (a0[0] + a1[0]) / deg
    o = jnp.dot(mean, Wl[...], preferred_element_type=jnp.float32)
    o = o + jnp.dot(h[...], Wr[...], preferred_element_type=jnp.float32)
    o = o + bl[...]
    o = jnp.maximum(o, 0.0)
    emb_ref[...] = o
    t = jnp.maximum(jnp.dot(o, W1[...], preferred_element_type=jnp.float32)
                    + b1[...], 0.0)
    logits = jnp.dot(t, W2[...], preferred_element_type=jnp.float32) + b2[...]
    m = jnp.max(logits, axis=1, keepdims=True)
    ex = jnp.exp(logits - m)
    lse = jnp.log(jnp.sum(ex, axis=1, keepdims=True)) + m
    out_ref[...] = logits - lse


def _tc_final(h, agg, d0, d1, Wl, Wr, bl, W1p, b1p, W2p, b2):
    P1 = W1p.shape[1]
    grid = (N // BROWS,)
    bs_rows = pl.BlockSpec((BROWS, H), lambda i: (i, 0))
    bs_a0 = pl.BlockSpec((1, BROWS, H), lambda i: (0, i, 0))
    bs_a1 = pl.BlockSpec((1, BROWS, H), lambda i: (1, i, 0))
    bs_col = pl.BlockSpec((BROWS, 1), lambda i: (i, 0))
    bs_w = pl.BlockSpec((H, H), lambda i: (0, 0))
    bs_v = pl.BlockSpec((1, H), lambda i: (0, 0))
    return pl.pallas_call(
        _tc_final_body,
        grid=grid,
        in_specs=[bs_rows, bs_a0, bs_a1, bs_col, bs_col, bs_w, bs_w, bs_v,
                  pl.BlockSpec((H, P1), lambda i: (0, 0)),
                  pl.BlockSpec((1, P1), lambda i: (0, 0)),
                  pl.BlockSpec((P1, OUT), lambda i: (0, 0)),
                  pl.BlockSpec((1, OUT), lambda i: (0, 0))],
        out_specs=[bs_rows, pl.BlockSpec((BROWS, OUT), lambda i: (i, 0))],
        out_shape=[jax.ShapeDtypeStruct((N, H), jnp.float32),
                   jax.ShapeDtypeStruct((N, OUT), jnp.float32)],
    )(h, agg, agg, d0, d1, Wl, Wr, bl, W1p, b1p, W2p, b2)


def kernel(x, edge_index, batch, Wl0, bl0, Wr0, Wl1, bl1, Wr1, Wl2, bl2, Wr2,
           g0, beta0, g1, beta1, W1, b1, W2, b2):
    src = edge_index[0]
    dst = edge_index[1]
    npad = E_PAD - E
    pad_src = jnp.arange(npad + SEG * CK, dtype=jnp.int32) % N
    srcm = jnp.concatenate([src, pad_src])
    pad_dst = N + (jnp.arange(npad, dtype=jnp.int32) % (ACC_ROWS - N))
    dstm = jnp.concatenate([dst, pad_dst]).reshape(TOT_CH, 1, CK)
    zrows = jnp.zeros((ACC_ROWS, H), jnp.float32)
    zdeg = jnp.zeros((ACC_ROWS,), jnp.float32)

    bl0_ = bl0.reshape(1, H)
    bl1_ = bl1.reshape(1, H)
    bl2_ = bl2.reshape(1, H)
    g0_ = g0.reshape(1, H)
    beta0_ = beta0.reshape(1, H)
    g1_ = g1.reshape(1, H)
    beta1_ = beta1.reshape(1, H)
    P1 = 64
    W1p = jnp.pad(W1, ((0, 0), (0, P1 - W1.shape[1])))
    b1p = jnp.pad(b1, (0, P1 - b1.shape[0])).reshape(1, P1)
    W2p = jnp.pad(W2, ((0, P1 - W2.shape[0]), (0, 0)))
    b2_ = b2.reshape(1, OUT)

    agg, deg0, deg1 = _segsum_deg(x, srcm, dstm, zrows, zdeg)
    d0 = deg0.reshape(ACC_ROWS, 1)
    d1 = deg1.reshape(ACC_ROWS, 1)
    h = _tc_layer(x, agg, d0, d1, Wl0, Wr0, bl0_, g0_, beta0_)
    (agg,) = _segsum(h, srcm, dstm, zrows, zdeg)
    h = _tc_layer(h, agg, d0, d1, Wl1, Wr1, bl1_, g1_, beta1_)
    (agg,) = _segsum(h, srcm, dstm, zrows, zdeg)
    emb, logp = _tc_final(h, agg, d0, d1, Wl2, Wr2, bl2_,
                          W1p, b1p, W2p, b2_)
    return (emb, logp)


# speedup vs baseline: 3.6681x; 1.0037x over previous
import functools

import jax
import jax.numpy as jnp
from jax import lax
from jax.experimental import pallas as pl
from jax.experimental.pallas import tpu as pltpu
from jax.experimental.pallas import tpu_sc as plsc

N = 10000
D = 128
H = 128
E = 320000
OUT = 40

NC = 2
NS = 16
NW = NC * NS

CK = 128
NBUF = 2
CH_BIG = 80
CH_SMALL = 80
BIG_ON_C0 = True
SEG = 80
TOT_CH = NS * (CH_BIG + CH_SMALL)
E_PAD = TOT_CH * CK
ACC_ROWS = 10240
ZROWS_W = ACC_ROWS // NS

BROWS = 2000


def _make_segsum(want_deg: bool):
    mesh = plsc.VectorSubcoreMesh(core_axis_name="c", subcore_axis_name="s")
    out_type = [jax.ShapeDtypeStruct((NC, ACC_ROWS, H), jnp.float32)]
    scratch = [
        pltpu.VMEM((SEG * CK,), jnp.int32),
        [pltpu.VMEM((1, CK), jnp.int32) for _ in range(NBUF)],
        [pltpu.VMEM((CK, H), jnp.float32) for _ in range(NBUF)],
        pltpu.VMEM_SHARED((ACC_ROWS, H), jnp.float32),
        [pltpu.SemaphoreType.DMA for _ in range(NBUF)],
        [pltpu.SemaphoreType.DMA for _ in range(NBUF)],
    ]
    if want_deg:
        out_type.append(jax.ShapeDtypeStruct((ACC_ROWS,), jnp.float32))
        out_type.append(jax.ShapeDtypeStruct((ACC_ROWS,), jnp.float32))
        scratch += [
            pltpu.VMEM((CK,), jnp.float32),
            pltpu.VMEM_SHARED((ACC_ROWS,), jnp.float32),
        ]

    def body(feat, srcm, dstm, zrows, zdeg, *rest):
        if want_deg:
            (agg_out, deg0_out, deg1_out, src_v, dsts, rows, acc_sh, sems,
             dsems, ones_v, deg_sh) = rest
        else:
            agg_out, src_v, dsts, rows, acc_sh, sems, dsems = rest
        c = lax.axis_index("c")
        s = lax.axis_index("s")
        z0 = s * ZROWS_W
        big = (c == 0) if BIG_ON_C0 else (c == 1)
        ch = jnp.where(big, CH_BIG, CH_SMALL)
        slab = jnp.where(big, s * CH_BIG, NS * CH_BIG + s * CH_SMALL)
        pltpu.sync_copy(zrows.at[pl.ds(z0, ZROWS_W)], acc_sh.at[pl.ds(z0, ZROWS_W)])
        if want_deg:
            pltpu.sync_copy(zdeg.at[pl.ds(z0, ZROWS_W)], deg_sh.at[pl.ds(z0, ZROWS_W)])
            for j in range(CK // 16):
                ones_v[pl.ds(j * 16, 16)] = jnp.ones((16,), jnp.float32)
        plsc.subcore_barrier()

        bufs = tuple(zip(rows, sems, dsts, dsems))

        def _consume(buf, dbuf):
            pltpu.sync_copy(buf, acc_sh.at[dbuf.at[0]], add=True)
            if want_deg:
                pltpu.sync_copy(ones_v, deg_sh.at[dbuf.at[0]], add=True)

        def run_segment(first, nch):
            pltpu.sync_copy(
                srcm.at[pl.ds(pl.multiple_of(first * CK, 128), SEG * CK)],
                src_v)

            def _start(j, buf, sem, dbuf, dsem):
                pltpu.async_copy(dstm.at[first + j], dbuf, dsem)
                pltpu.async_copy(feat.at[src_v.at[pl.ds(j * CK, CK)]], buf, sem)

            def _drain(buf, sem, dbuf, dsem):
                pltpu.make_async_copy(dstm.at[0], dbuf, dsem).wait()
                pltpu.make_async_copy(
                    feat.at[src_v.at[pl.ds(0, CK)]], buf, sem).wait()

            for b, (buf, sem, dbuf, dsem) in enumerate(bufs):
                _start(b, buf, sem, dbuf, dsem)

            def step(g, carry):
                for b, (buf, sem, dbuf, dsem) in enumerate(bufs):
                    j = g * NBUF + b
                    _drain(buf, sem, dbuf, dsem)
                    _consume(buf, dbuf)
                    _start(j + NBUF, buf, sem, dbuf, dsem)
                return carry

            lax.fori_loop(0, nch // NBUF - 1, step, 0)
            for b, (buf, sem, dbuf, dsem) in enumerate(bufs):
                _drain(buf, sem, dbuf, dsem)
                _consume(buf, dbuf)

        run_segment(slab, jnp.minimum(ch, SEG))

        @pl.when(ch > SEG)
        def _():
            run_segment(slab + SEG, ch - SEG)

        plsc.subcore_barrier()
        pltpu.sync_copy(acc_sh.at[pl.ds(z0, ZROWS_W)], agg_out.at[c, pl.ds(z0, ZROWS_W)])
        if want_deg:
            @pl.when(c == 0)
            def _():
                pltpu.sync_copy(deg_sh.at[pl.ds(z0, ZROWS_W)],
                                deg0_out.at[pl.ds(z0, ZROWS_W)])

            @pl.when(c == 1)
            def _():
                pltpu.sync_copy(deg_sh.at[pl.ds(z0, ZROWS_W)],
                                deg1_out.at[pl.ds(z0, ZROWS_W)])

    return pl.kernel(body, out_type=tuple(out_type), mesh=mesh,
                     scratch_types=tuple(scratch))


_segsum_deg = _make_segsum(True)
_segsum = _make_segsum(False)


def _tc_root_body(h, Wr, bl, r_ref):
    r_ref[...] = jnp.dot(h[...], Wr[...],
                         preferred_element_type=jnp.float32) + bl[...]


def _tc_root(h, Wr, bl):
    grid = (N // BROWS,)
    bs_rows = pl.BlockSpec((BROWS, H), lambda i: (i, 0))
    return pl.pallas_call(
        _tc_root_body,
        grid=grid,
        in_specs=[bs_rows, pl.BlockSpec((H, H), lambda i: (0, 0)),
                  pl.BlockSpec((1, H), lambda i: (0, 0))],
        out_specs=bs_rows,
        out_shape=jax.ShapeDtypeStruct((N, H), jnp.float32),
    )(h, Wr, bl)


def _tc_layer_body(r, a0, a1, d0, d1, Wl, g, beta, o_ref):
    deg = jnp.maximum(d0[...] + d1[...], 1.0)
    mean = (a0[0] + a1[0]) / deg
    o = jnp.dot(mean, Wl[...], preferred_element_type=jnp.float32) + r[...]
    o = jnp.maximum(o, 0.0)
    mu = jnp.mean(o, axis=1, keepdims=True)
    var = jnp.mean((o - mu) ** 2, axis=1, keepdims=True)
    o_ref[...] = (o - mu) * lax.rsqrt(var + 1e-5) * g[...] + beta[...]


def _tc_layer(r, agg, d0, d1, Wl, g, beta):
    grid = (N // BROWS,)
    bs_rows = pl.BlockSpec((BROWS, H), lambda i: (i, 0))
    bs_a0 = pl.BlockSpec((1, BROWS, H), lambda i: (0, i, 0))
    bs_a1 = pl.BlockSpec((1, BROWS, H), lambda i: (1, i, 0))
    bs_col = pl.BlockSpec((BROWS, 1), lambda i: (i, 0))
    bs_w = pl.BlockSpec((H, H), lambda i: (0, 0))
    bs_v = pl.BlockSpec((1, H), lambda i: (0, 0))
    return pl.pallas_call(
        _tc_layer_body,
        grid=grid,
        in_specs=[bs_rows, bs_a0, bs_a1, bs_col, bs_col, bs_w, bs_v, bs_v],
        out_specs=bs_rows,
        out_shape=jax.ShapeDtypeStruct((N, H), jnp.float32),
    )(r, agg, agg, d0, d1, Wl, g, beta)


def _tc_final_body(r, a0, a1, d0, d1, Wl, W1, b1, W2, b2,
                   emb_ref, out_ref):
    deg = jnp.maximum(d0[...] + d1[...], 1.0)
    mean = (a0[0] + a1[0]) / deg
    o = jnp.dot(mean, Wl[...], preferred_element_type=jnp.float32) + r[...]
    o = jnp.maximum(o, 0.0)
    emb_ref[...] = o
    t = jnp.maximum(jnp.dot(o, W1[...], preferred_element_type=jnp.float32)
                    + b1[...], 0.0)
    logits = jnp.dot(t, W2[...], preferred_element_type=jnp.float32) + b2[...]
    m = jnp.max(logits, axis=1, keepdims=True)
    ex = jnp.exp(logits - m)
    lse = jnp.log(jnp.sum(ex, axis=1, keepdims=True)) + m
    out_ref[...] = logits - lse


def _tc_final(r, agg, d0, d1, Wl, W1p, b1p, W2p, b2):
    P1 = W1p.shape[1]
    grid = (N // BROWS,)
    bs_rows = pl.BlockSpec((BROWS, H), lambda i: (i, 0))
    bs_a0 = pl.BlockSpec((1, BROWS, H), lambda i: (0, i, 0))
    bs_a1 = pl.BlockSpec((1, BROWS, H), lambda i: (1, i, 0))
    bs_col = pl.BlockSpec((BROWS, 1), lambda i: (i, 0))
    bs_w = pl.BlockSpec((H, H), lambda i: (0, 0))
    return pl.pallas_call(
        _tc_final_body,
        grid=grid,
        in_specs=[bs_rows, bs_a0, bs_a1, bs_col, bs_col, bs_w,
                  pl.BlockSpec((H, P1), lambda i: (0, 0)),
                  pl.BlockSpec((1, P1), lambda i: (0, 0)),
                  pl.BlockSpec((P1, OUT), lambda i: (0, 0)),
                  pl.BlockSpec((1, OUT), lambda i: (0, 0))],
        out_specs=[bs_rows, pl.BlockSpec((BROWS, OUT), lambda i: (i, 0))],
        out_shape=[jax.ShapeDtypeStruct((N, H), jnp.float32),
                   jax.ShapeDtypeStruct((N, OUT), jnp.float32)],
    )(r, agg, agg, d0, d1, Wl, W1p, b1p, W2p, b2)


def kernel(x, edge_index, batch, Wl0, bl0, Wr0, Wl1, bl1, Wr1, Wl2, bl2, Wr2,
           g0, beta0, g1, beta1, W1, b1, W2, b2):
    src = edge_index[0]
    dst = edge_index[1]
    npad = E_PAD - E
    pad_src = jnp.arange(npad + SEG * CK, dtype=jnp.int32) % N
    srcm = jnp.concatenate([src, pad_src])
    pad_dst = N + (jnp.arange(npad, dtype=jnp.int32) % (ACC_ROWS - N))
    dstm = jnp.concatenate([dst, pad_dst]).reshape(TOT_CH, 1, CK)
    zrows = jnp.zeros((ACC_ROWS, H), jnp.float32)
    zdeg = jnp.zeros((ACC_ROWS,), jnp.float32)

    bl0_ = bl0.reshape(1, H)
    bl1_ = bl1.reshape(1, H)
    bl2_ = bl2.reshape(1, H)
    g0_ = g0.reshape(1, H)
    beta0_ = beta0.reshape(1, H)
    g1_ = g1.reshape(1, H)
    beta1_ = beta1.reshape(1, H)
    P1 = 64
    W1p = jnp.pad(W1, ((0, 0), (0, P1 - W1.shape[1])))
    b1p = jnp.pad(b1, (0, P1 - b1.shape[0])).reshape(1, P1)
    W2p = jnp.pad(W2, ((0, P1 - W2.shape[0]), (0, 0)))
    b2_ = b2.reshape(1, OUT)

    agg, deg0, deg1 = _segsum_deg(x, srcm, dstm, zrows, zdeg)
    r = _tc_root(x, Wr0, bl0_)
    d0 = deg0.reshape(ACC_ROWS, 1)
    d1 = deg1.reshape(ACC_ROWS, 1)
    h = _tc_layer(r, agg, d0, d1, Wl0, g0_, beta0_)
    (agg,) = _segsum(h, srcm, dstm, zrows, zdeg)
    r = _tc_root(h, Wr1, bl1_)
    h = _tc_layer(r, agg, d0, d1, Wl1, g1_, beta1_)
    (agg,) = _segsum(h, srcm, dstm, zrows, zdeg)
    r = _tc_root(h, Wr2, bl2_)
    emb, logp = _tc_final(r, agg, d0, d1, Wl2, W1p, b1p, W2p, b2_)
    return (emb, logp)
